# baseline (speedup 1.0000x reference)
import functools

import jax
import jax.numpy as jnp
from jax import lax
from jax.experimental import pallas as pl
from jax.experimental.pallas import tpu as pltpu
from jax.experimental.pallas import tpu_sc as plsc

B = 4096
F = 26
D = 32
TR = 8
NW = 32
RPW = B // NW
NPW = RPW * F
GCHUNK = 104
NG = NPW // GCHUNK
RPC = GCHUNK // F


def _fm_body(ids_hbm, vals_hbm, emb3_hbm, btab_hbm, bias_hbm, out_hbm,
             idx_v, vals_v, tile_v, brow_v, out_v, bias_s,
             sem, bsem):
    nc = 2
    wid = lax.axis_index("s") * nc + lax.axis_index("c")

    pltpu.sync_copy(ids_hbm.at[pl.ds(wid * NPW, NPW)],
                    idx_v.at[pl.ds(0, NPW)])
    pltpu.sync_copy(vals_hbm.at[pl.ds(wid * NPW, NPW)],
                    vals_v.at[pl.ds(0, NPW)])
    pltpu.sync_copy(bias_hbm, bias_s.at[pl.ds(0, 1)])

    bcopies = []
    for j in range(NG):
        bcopies.append(pltpu.async_copy(
            btab_hbm.at[idx_v.at[pl.ds(j * GCHUNK, GCHUNK)]],
            brow_v.at[pl.ds(j * GCHUNK, GCHUNK)], bsem))
    for c in bcopies:
        c.wait()

    bias0 = bias_s[pl.ds(0, 16)][0]
    lane = lax.iota(jnp.int32, 16)
    tail_mask = lane < (F - 16)
    zeros = jnp.zeros((16,), jnp.float32)

    def chunk_body(c, carry):
        base = c * GCHUNK
        pltpu.async_copy(
            emb3_hbm.at[idx_v.at[pl.ds(base, GCHUNK)]],
            tile_v, sem).wait()

        def row_body(i, cr):
            off = base + i * F
            v0 = vals_v[pl.ds(off, 16)]
            v1 = vals_v[pl.ds(off + 16, 16)]
            b0 = brow_v[pl.ds(off, 16)]
            b1 = brow_v[pl.ds(off + 16, 16)]
            s0 = zeros
            s1 = zeros
            q0 = zeros
            q1 = zeros
            for f in range(F):
                v = v0[f] if f < 16 else v1[f - 16]
                s = i * F + f
                t0 = tile_v[s, pl.ds(0, 16)] * v
                t1 = tile_v[s, pl.ds(16, 16)] * v
                s0 = s0 + t0
                s1 = s1 + t1
                q0 = q0 + t0 * t0
                q1 = q1 + t1 * t1
            bacc = jnp.sum(b0 * v0 + jnp.where(tail_mask, b1 * v1, zeros))
            red = jnp.sum(s0 * s0 - q0 + s1 * s1 - q1) * (1.0 / 64.0)
            pred = jnp.full((16,), red + bacc + bias0, jnp.float32)
            plsc.store_scatter(out_v,
                               [jnp.full((16,), c * RPC + i, jnp.int32)],
                               pred, mask=lane == 0)
            return cr

        lax.fori_loop(0, RPC, row_body, 0)
        return carry

    lax.fori_loop(0, NG, chunk_body, 0)
    pltpu.sync_copy(out_v, out_hbm.at[pl.ds(wid * RPW, RPW)])


def kernel(feature_ids, feature_vals, emb_table, bias_table, bias):
    ids_flat = feature_ids.reshape(B * F)
    vals_flat = feature_vals.reshape(B * F)
    btab_flat = bias_table.reshape(-1)
    emb3 = jnp.pad(emb_table, ((0, 0), (0, 128 - D)))

    mesh = plsc.VectorSubcoreMesh(core_axis_name="c", subcore_axis_name="s")
    k = functools.partial(
        pl.kernel,
        out_type=jax.ShapeDtypeStruct((B,), jnp.float32),
        mesh=mesh,
        compiler_params=pltpu.CompilerParams(
            needs_layout_passes=False, use_tc_tiling_on_sc=True),
        scratch_types=[
            pltpu.VMEM((NPW + 16,), jnp.int32),
            pltpu.VMEM((NPW + 16,), jnp.float32),
            pltpu.VMEM((GCHUNK, 128), jnp.float32),
            pltpu.VMEM((NPW + 16,), jnp.float32),
            pltpu.VMEM((RPW,), jnp.float32),
            pltpu.VMEM((16,), jnp.float32),
            pltpu.SemaphoreType.DMA,
            pltpu.SemaphoreType.DMA,
        ],
    )(_fm_body)
    return k(ids_flat, vals_flat, emb3, btab_flat, bias)


# speedup vs baseline: 1.0461x; 1.0461x over previous
import functools

import jax
import jax.numpy as jnp
from jax import lax
from jax.experimental import pallas as pl
from jax.experimental.pallas import tpu as pltpu
from jax.experimental.pallas import tpu_sc as plsc

B = 4096
F = 26
D = 32
NW = 32
RPW = B // NW
NPW = RPW * F
GCHUNK = 104
NG = NPW // GCHUNK


def _fm_body(ids_hbm, vals_hbm, emb_hbm, btab_hbm, bias_hbm, out_hbm,
             idx_v, vals_v, rows_v, brow_v, out_v, bias_s, sem):
    nc = 2
    wid = lax.axis_index("s") * nc + lax.axis_index("c")

    pltpu.sync_copy(ids_hbm.at[pl.ds(wid * NG, NG), :], idx_v)
    pltpu.sync_copy(vals_hbm.at[pl.ds(wid * NPW, NPW)],
                    vals_v.at[pl.ds(0, NPW)])
    pltpu.sync_copy(bias_hbm, bias_s.at[pl.ds(0, 1)])

    copies = []
    for j in range(NG):
        copies.append(pltpu.async_copy(
            emb_hbm.at[idx_v.at[j]],
            rows_v.at[pl.ds(j * GCHUNK, GCHUNK)], sem))
        copies.append(pltpu.async_copy(
            btab_hbm.at[idx_v.at[j]],
            brow_v.at[pl.ds(j * GCHUNK, GCHUNK)], sem))
    for c in copies:
        c.wait()

    bias0 = bias_s[pl.ds(0, 16)][0]
    lane = lax.iota(jnp.int32, 16)
    zlane = jnp.zeros((16,), jnp.int32)
    tail_mask = lane < (F - 16)
    zeros = jnp.zeros((16,), jnp.float32)

    def row_body(i, carry):
        off = i * F
        v0 = vals_v[pl.ds(off, 16)]
        v1 = vals_v[pl.ds(off + 16, 16)]
        b0 = brow_v[pl.ds(off, 16)]
        b1 = brow_v[pl.ds(off + 16, 16)]
        s0 = zeros
        s1 = zeros
        q0 = zeros
        q1 = zeros
        for f in range(F):
            v = v0[f] if f < 16 else v1[f - 16]
            t0 = rows_v[off + f, pl.ds(0, 16)] * v
            t1 = rows_v[off + f, pl.ds(16, 16)] * v
            s0 = s0 + t0
            s1 = s1 + t1
            q0 = q0 + t0 * t0
            q1 = q1 + t1 * t1
        bacc = jnp.sum(b0 * v0 + jnp.where(tail_mask, b1 * v1, zeros))
        red = jnp.sum(s0 * s0 - q0 + s1 * s1 - q1) * (1.0 / 64.0)
        pred = jnp.full((16,), red + bacc + bias0, jnp.float32)
        plsc.store_scatter(out_v, [jnp.full((16,), i, jnp.int32)], pred,
                           mask=lane == 0)
        return carry

    lax.fori_loop(0, RPW, row_body, 0)
    pltpu.sync_copy(out_v, out_hbm.at[pl.ds(wid * RPW, RPW)])


def kernel(feature_ids, feature_vals, emb_table, bias_table, bias):
    ids2d = feature_ids.reshape(B * F // GCHUNK, GCHUNK)
    vals_flat = feature_vals.reshape(B * F)
    btab_flat = bias_table.reshape(-1)

    mesh = plsc.VectorSubcoreMesh(core_axis_name="c", subcore_axis_name="s")
    k = functools.partial(
        pl.kernel,
        out_type=jax.ShapeDtypeStruct((B,), jnp.float32),
        mesh=mesh,
        compiler_params=pltpu.CompilerParams(
            needs_layout_passes=False, use_tc_tiling_on_sc=False),
        scratch_types=[
            pltpu.VMEM((NG, GCHUNK), jnp.int32),
            pltpu.VMEM((NPW + 16,), jnp.float32),
            pltpu.VMEM((NPW, D), jnp.float32),
            pltpu.VMEM((NPW + 16,), jnp.float32),
            pltpu.VMEM((RPW,), jnp.float32),
            pltpu.VMEM((16,), jnp.float32),
            pltpu.SemaphoreType.DMA,
        ],
    )(_fm_body)
    return k(ids2d, vals_flat, emb_table, btab_flat, bias)


# speedup vs baseline: 1.4459x; 1.3822x over previous
import functools

import jax
import jax.numpy as jnp
from jax import lax
from jax.experimental import pallas as pl
from jax.experimental.pallas import tpu as pltpu
from jax.experimental.pallas import tpu_sc as plsc

B = 4096
F = 26
D = 32
TR = 8
NW = 32
RPW = B // NW
NPW = RPW * F
GCHUNK = 52
NG = NPW // GCHUNK
RPC = GCHUNK // F


def _fm_body(ids_hbm, vals_hbm, emb3_hbm, btab_hbm, bias_hbm, out_hbm,
             idx_v, vals_v, tile_a, tile_b, brow_v, out_v, bias_s,
             sem_a, sem_b, bsem):
    nc = 2
    wid = lax.axis_index("s") * nc + lax.axis_index("c")

    pltpu.sync_copy(ids_hbm.at[pl.ds(wid * NPW, NPW)],
                    idx_v.at[pl.ds(0, NPW)])
    pltpu.sync_copy(vals_hbm.at[pl.ds(wid * NPW, NPW)],
                    vals_v.at[pl.ds(0, NPW)])
    pltpu.sync_copy(bias_hbm, bias_s.at[pl.ds(0, 1)])

    bcopies = []
    for j in range(NPW // 104):
        bcopies.append(pltpu.async_copy(
            btab_hbm.at[idx_v.at[pl.ds(j * 104, 104)]],
            brow_v.at[pl.ds(j * 104, 104)], bsem))
    for c in bcopies:
        c.wait()

    bias0 = bias_s[pl.ds(0, 16)][0]
    lane = lax.iota(jnp.int32, 16)
    tail_mask = lane < (F - 16)
    zeros = jnp.zeros((16,), jnp.float32)

    def fire(c, tile_v, sem):
        base = c * GCHUNK
        copies = []
        for s in range(GCHUNK):
            idvec = idx_v[pl.ds(base + (s // 16) * 16, 16)]
            blk = lax.shift_right_logical(idvec[s % 16], 3)
            copies.append(pltpu.async_copy(
                emb3_hbm.at[blk], tile_v.at[s], sem))
        return copies

    def compute(c, tile_v):
        base = c * GCHUNK

        def row_body(i, cr):
            off = base + i * F
            v0 = vals_v[pl.ds(off, 16)]
            v1 = vals_v[pl.ds(off + 16, 16)]
            id0 = idx_v[pl.ds(off, 16)]
            id1 = idx_v[pl.ds(off + 16, 16)]
            b0 = brow_v[pl.ds(off, 16)]
            b1 = brow_v[pl.ds(off + 16, 16)]
            s0 = zeros
            s1 = zeros
            q0 = zeros
            q1 = zeros
            for f in range(F):
                if f < 16:
                    v = v0[f]
                    rsub = id0[f]
                else:
                    v = v1[f - 16]
                    rsub = id1[f - 16]
                rsub = lax.bitwise_and(rsub, 7)
                s = i * F + f
                t0 = tile_v[s, rsub, pl.ds(0, 16)] * v
                t1 = tile_v[s, rsub, pl.ds(16, 16)] * v
                s0 = s0 + t0
                s1 = s1 + t1
                q0 = q0 + t0 * t0
                q1 = q1 + t1 * t1
            bacc = jnp.sum(b0 * v0 + jnp.where(tail_mask, b1 * v1, zeros))
            red = jnp.sum(s0 * s0 - q0 + s1 * s1 - q1) * (1.0 / 64.0)
            pred = jnp.full((16,), red + bacc + bias0, jnp.float32)
            plsc.store_scatter(out_v,
                               [jnp.full((16,), c * RPC + i, jnp.int32)],
                               pred, mask=lane == 0)
            return cr

        lax.fori_loop(0, RPC, row_body, 0)

    def pair_body(p, carry):
        ca = 2 * p
        cb = 2 * p + 1
        copies_a = fire(ca, tile_a, sem_a)
        copies_b = fire(cb, tile_b, sem_b)
        for cp in copies_a:
            cp.wait()
        compute(ca, tile_a)
        for cp in copies_b:
            cp.wait()
        compute(cb, tile_b)
        return carry

    lax.fori_loop(0, NG // 2, pair_body, 0)
    pltpu.sync_copy(out_v, out_hbm.at[pl.ds(wid * RPW, RPW)])


def kernel(feature_ids, feature_vals, emb_table, bias_table, bias):
    ids_flat = feature_ids.reshape(B * F)
    vals_flat = feature_vals.reshape(B * F)
    btab_flat = bias_table.reshape(-1)
    emb3 = emb_table.reshape(1000000 // TR, TR, D)

    mesh = plsc.VectorSubcoreMesh(core_axis_name="c", subcore_axis_name="s")
    k = functools.partial(
        pl.kernel,
        out_type=jax.ShapeDtypeStruct((B,), jnp.float32),
        mesh=mesh,
        compiler_params=pltpu.CompilerParams(
            needs_layout_passes=False, use_tc_tiling_on_sc=True),
        scratch_types=[
            pltpu.VMEM((NPW + 16,), jnp.int32),
            pltpu.VMEM((NPW + 16,), jnp.float32),
            pltpu.VMEM((GCHUNK, TR, D), jnp.float32),
            pltpu.VMEM((GCHUNK, TR, D), jnp.float32),
            pltpu.VMEM((NPW + 16,), jnp.float32),
            pltpu.VMEM((RPW,), jnp.float32),
            pltpu.VMEM((16,), jnp.float32),
            pltpu.SemaphoreType.DMA,
            pltpu.SemaphoreType.DMA,
            pltpu.SemaphoreType.DMA,
        ],
    )(_fm_body)
    return k(ids_flat, vals_flat, emb3, btab_flat, bias)


# speedup vs baseline: 2.1576x; 1.4922x over previous
import functools

import jax
import jax.numpy as jnp
from jax import lax
from jax.experimental import pallas as pl
from jax.experimental.pallas import tpu as pltpu
from jax.experimental.pallas import tpu_sc as plsc

B = 4096
F = 26
D = 32
TR = 8
NW = 32
RPW = B // NW
NPW = RPW * F
GCHUNK = 52
NG = NPW // GCHUNK
RPC = GCHUNK // F


def _fm_body(ids_hbm, vals_hbm, emb3_hbm, btab_hbm, bias_hbm, out_hbm,
             idx_v, vals_v, tile_a, tile_b, brow_v, out_v, bias_s,
             sem_a, sem_b, bsem):
    nc = 2
    wid = lax.axis_index("s") * nc + lax.axis_index("c")

    pltpu.sync_copy(ids_hbm.at[pl.ds(wid * NPW, NPW)],
                    idx_v.at[pl.ds(0, NPW)])
    pltpu.sync_copy(vals_hbm.at[pl.ds(wid * NPW, NPW)],
                    vals_v.at[pl.ds(0, NPW)])
    pltpu.sync_copy(bias_hbm, bias_s.at[pl.ds(0, 1)])

    bcopies = []
    for j in range(NPW // 104):
        bcopies.append(pltpu.async_copy(
            btab_hbm.at[idx_v.at[pl.ds(j * 104, 104)]],
            brow_v.at[pl.ds(j * 104, 104)], bsem))
    for c in bcopies:
        c.wait()

    bias0 = bias_s[pl.ds(0, 16)][0]
    lane = lax.iota(jnp.int32, 16)
    tail_mask = lane < (F - 16)
    zeros = jnp.zeros((16,), jnp.float32)

    def fire(c, tile_v, sem):
        base = c * GCHUNK
        copies = []
        for s in range(GCHUNK):
            idvec = idx_v[pl.ds(base + (s // 16) * 16, 16)]
            tid = idvec[s % 16]
            blk = lax.shift_right_logical(tid, 3)
            rsub = lax.bitwise_and(tid, 7)
            copies.append(pltpu.async_copy(
                emb3_hbm.at[blk, rsub], tile_v.at[s], sem))
        return copies

    def compute(c, tile_v):
        base = c * GCHUNK

        def row_body(i, cr):
            off = base + i * F
            v0 = vals_v[pl.ds(off, 16)]
            v1 = vals_v[pl.ds(off + 16, 16)]
            b0 = brow_v[pl.ds(off, 16)]
            b1 = brow_v[pl.ds(off + 16, 16)]
            s0 = zeros
            s1 = zeros
            q0 = zeros
            q1 = zeros
            for f in range(F):
                v = v0[f] if f < 16 else v1[f - 16]
                s = i * F + f
                t0 = tile_v[s, pl.ds(0, 16)] * v
                t1 = tile_v[s, pl.ds(16, 16)] * v
                s0 = s0 + t0
                s1 = s1 + t1
                q0 = q0 + t0 * t0
                q1 = q1 + t1 * t1
            bacc = jnp.sum(b0 * v0 + jnp.where(tail_mask, b1 * v1, zeros))
            red = jnp.sum(s0 * s0 - q0 + s1 * s1 - q1) * (1.0 / 64.0)
            pred = jnp.full((16,), red + bacc + bias0, jnp.float32)
            plsc.store_scatter(out_v,
                               [jnp.full((16,), c * RPC + i, jnp.int32)],
                               pred, mask=lane == 0)
            return cr

        lax.fori_loop(0, RPC, row_body, 0)

    def pair_body(p, carry):
        ca = 2 * p
        cb = 2 * p + 1
        copies_a = fire(ca, tile_a, sem_a)
        copies_b = fire(cb, tile_b, sem_b)
        for cp in copies_a:
            cp.wait()
        compute(ca, tile_a)
        for cp in copies_b:
            cp.wait()
        compute(cb, tile_b)
        return carry

    lax.fori_loop(0, NG // 2, pair_body, 0)
    pltpu.sync_copy(out_v, out_hbm.at[pl.ds(wid * RPW, RPW)])


def kernel(feature_ids, feature_vals, emb_table, bias_table, bias):
    ids_flat = feature_ids.reshape(B * F)
    vals_flat = feature_vals.reshape(B * F)
    btab_flat = bias_table.reshape(-1)
    emb3 = emb_table.reshape(1000000 // TR, TR, D)

    mesh = plsc.VectorSubcoreMesh(core_axis_name="c", subcore_axis_name="s")
    k = functools.partial(
        pl.kernel,
        out_type=jax.ShapeDtypeStruct((B,), jnp.float32),
        mesh=mesh,
        compiler_params=pltpu.CompilerParams(
            needs_layout_passes=False, use_tc_tiling_on_sc=True),
        scratch_types=[
            pltpu.VMEM((NPW + 16,), jnp.int32),
            pltpu.VMEM((NPW + 16,), jnp.float32),
            pltpu.VMEM((GCHUNK, D), jnp.float32),
            pltpu.VMEM((GCHUNK, D), jnp.float32),
            pltpu.VMEM((NPW + 16,), jnp.float32),
            pltpu.VMEM((RPW,), jnp.float32),
            pltpu.VMEM((16,), jnp.float32),
            pltpu.SemaphoreType.DMA,
            pltpu.SemaphoreType.DMA,
            pltpu.SemaphoreType.DMA,
        ],
    )(_fm_body)
    return k(ids_flat, vals_flat, emb3, btab_flat, bias)
